# Initial kernel scaffold; baseline (speedup 1.0000x reference)
import functools

import jax
import jax.numpy as jnp
from jax import lax
from jax.experimental import pallas as pl
from jax.experimental.pallas import tpu as pltpu
from jax.experimental.pallas import tpu_sc as plsc

NUM_HEADS = 16
VOCAB = 65536
HF = 2
BATCH = 16384
MLP_DIM = 64
OUT_DIM = 3
IN_FEATS = NUM_HEADS * HF

NC = 2
NS = 16
NW = NC * NS
LANES = 16

TOTAL = BATCH * NUM_HEADS
PER_W = TOTAL // NW
CHUNK = 128
NCHUNK = PER_W // CHUNK


def _sc_gather_body(table_hbm, idx_hbm, out_hbm, idx_v, rows_v, sem):
  wid = lax.axis_index("s") * NC + lax.axis_index("c")
  row0 = wid * NCHUNK

  pltpu.sync_copy(idx_hbm.at[pl.ds(row0, NCHUNK)], idx_v)

  offs = lax.iota(jnp.int32, (LANES,)) * VOCAB

  def fire(c, carry):
    for j in range(CHUNK // LANES):
      sl = pl.ds(j * LANES, LANES)
      idx_v[c, sl] = idx_v[c, sl] + offs
    pltpu.async_copy(table_hbm.at[idx_v.at[c]],
                     rows_v.at[pl.ds(c * CHUNK, CHUNK)], sem)
    return carry

  lax.fori_loop(0, NCHUNK, fire, 0)

  def drain(c, carry):
    pltpu.make_async_copy(table_hbm.at[pl.ds(0, CHUNK)],
                          rows_v.at[pl.ds(0, CHUNK)], sem).wait()
    return carry

  lax.fori_loop(0, NCHUNK, drain, 0)

  pltpu.sync_copy(rows_v, out_hbm.at[pl.ds(wid * PER_W, PER_W)])


_sc_gather = functools.partial(
    pl.kernel,
    out_type=jax.ShapeDtypeStruct((TOTAL, HF), jnp.float32),
    mesh=plsc.VectorSubcoreMesh(core_axis_name="c", subcore_axis_name="s"),
    scratch_types=[
        pltpu.VMEM((NCHUNK, CHUNK), jnp.int32),
        pltpu.VMEM((PER_W, HF), jnp.float32),
        pltpu.SemaphoreType.DMA,
    ],
)(_sc_gather_body)


def _mlp_body(h_ref, w1, b1, w2, b2, w3, b3, w4, b4, o_ref):
  x = h_ref[...]
  x = jnp.maximum(
      jnp.dot(x, w1[...], preferred_element_type=jnp.float32) + b1[...], 0.0)
  x = jnp.maximum(
      jnp.dot(x, w2[...], preferred_element_type=jnp.float32) + b2[...], 0.0)
  x = jnp.maximum(
      jnp.dot(x, w3[...], preferred_element_type=jnp.float32) + b3[...], 0.0)
  o_ref[...] = jnp.dot(x, w4[...], preferred_element_type=jnp.float32) + b4[...]


BBLK = 2048


def _mlp(h, W1, b1, W2, b2, W3, b3, W4, b4):
  full = lambda i: (0, 0)
  return pl.pallas_call(
      _mlp_body,
      grid=(BATCH // BBLK,),
      in_specs=[
          pl.BlockSpec((BBLK, IN_FEATS), lambda i: (i, 0)),
          pl.BlockSpec((IN_FEATS, MLP_DIM), full),
          pl.BlockSpec((1, MLP_DIM), full),
          pl.BlockSpec((MLP_DIM, MLP_DIM), full),
          pl.BlockSpec((1, MLP_DIM), full),
          pl.BlockSpec((MLP_DIM, MLP_DIM), full),
          pl.BlockSpec((1, MLP_DIM), full),
          pl.BlockSpec((MLP_DIM, OUT_DIM), full),
          pl.BlockSpec((1, OUT_DIM), full),
      ],
      out_specs=pl.BlockSpec((BBLK, OUT_DIM), lambda i: (i, 0)),
      out_shape=jax.ShapeDtypeStruct((BATCH, OUT_DIM), jnp.float32),
  )(h, W1, b1, W2, b2, W3, b3, W4, b4)


@jax.jit
def kernel(input, tables, W1, b1, W2, b2, W3, b3, W4, b4):
  idx2d = input.reshape(TOTAL // CHUNK, CHUNK)
  flat_tables = tables.reshape(NUM_HEADS * VOCAB, HF)
  emb = _sc_gather(flat_tables, idx2d)
  h = emb.reshape(BATCH, IN_FEATS)
  return _mlp(h, W1, b1.reshape(1, MLP_DIM), W2, b2.reshape(1, MLP_DIM),
              W3, b3.reshape(1, MLP_DIM), W4, b4.reshape(1, OUT_DIM))


# speedup vs baseline: 20.2874x; 20.2874x over previous
import functools

import jax
import jax.numpy as jnp
from jax import lax
from jax.experimental import pallas as pl
from jax.experimental.pallas import tpu as pltpu
from jax.experimental.pallas import tpu_sc as plsc

NUM_HEADS = 16
VOCAB = 65536
HF = 2
BATCH = 16384
MLP_DIM = 64
OUT_DIM = 3
IN_FEATS = NUM_HEADS * HF

NC = 2
NS = 16
NW = NC * NS
LANES = 16


def _sc_gather_body(table_hbm, idx_hbm, out_hbm, tbl_v, idx_v, out_v, sem):
  wid = lax.axis_index("s") * NC + lax.axis_index("c")
  head = wid // 2

  pltpu.sync_copy(table_hbm.at[wid], tbl_v)
  pltpu.sync_copy(idx_hbm.at[head], idx_v)

  @pl.loop(0, BATCH // LANES)
  def step(i):
    sl = pl.ds(i * LANES, LANES)
    out_v[sl] = plsc.load_gather(tbl_v, [idx_v[sl]])

  pltpu.sync_copy(out_v, out_hbm.at[wid])


_sc_gather = functools.partial(
    pl.kernel,
    out_type=jax.ShapeDtypeStruct((NW, BATCH), jnp.float32),
    mesh=plsc.VectorSubcoreMesh(core_axis_name="c", subcore_axis_name="s"),
    scratch_types=[
        pltpu.VMEM((VOCAB,), jnp.float32),
        pltpu.VMEM((BATCH,), jnp.int32),
        pltpu.VMEM((BATCH,), jnp.float32),
        pltpu.SemaphoreType.DMA,
    ],
    compiler_params=pltpu.CompilerParams(
        use_tc_tiling_on_sc=False, needs_layout_passes=False),
)(_sc_gather_body)


def _mlp_body(h_ref, w1, b1, w2, b2, w3, b3, w4, b4, o_ref):
  x = h_ref[...]
  x = jnp.maximum(
      jnp.dot(x, w1[...], preferred_element_type=jnp.float32) + b1[...], 0.0)
  x = jnp.maximum(
      jnp.dot(x, w2[...], preferred_element_type=jnp.float32) + b2[...], 0.0)
  x = jnp.maximum(
      jnp.dot(x, w3[...], preferred_element_type=jnp.float32) + b3[...], 0.0)
  o_ref[...] = jnp.dot(x, w4[...], preferred_element_type=jnp.float32) + b4[...]


BBLK = 2048


def _mlp(h, W1, b1, W2, b2, W3, b3, W4, b4):
  full = lambda i: (0, 0)
  return pl.pallas_call(
      _mlp_body,
      grid=(BATCH // BBLK,),
      in_specs=[
          pl.BlockSpec((BBLK, IN_FEATS), lambda i: (i, 0)),
          pl.BlockSpec((IN_FEATS, MLP_DIM), full),
          pl.BlockSpec((1, MLP_DIM), full),
          pl.BlockSpec((MLP_DIM, MLP_DIM), full),
          pl.BlockSpec((1, MLP_DIM), full),
          pl.BlockSpec((MLP_DIM, MLP_DIM), full),
          pl.BlockSpec((1, MLP_DIM), full),
          pl.BlockSpec((MLP_DIM, OUT_DIM), full),
          pl.BlockSpec((1, OUT_DIM), full),
      ],
      out_specs=pl.BlockSpec((BBLK, OUT_DIM), lambda i: (i, 0)),
      out_shape=jax.ShapeDtypeStruct((BATCH, OUT_DIM), jnp.float32),
  )(h, W1, b1, W2, b2, W3, b3, W4, b4)


@jax.jit
def kernel(input, tables, W1, b1, W2, b2, W3, b3, W4, b4):
  tables_t = tables.transpose(0, 2, 1).reshape(NW, VOCAB)
  idx_t = input.T
  emb = _sc_gather(tables_t, idx_t)
  h = emb.reshape(NUM_HEADS, HF, BATCH).transpose(2, 0, 1).reshape(
      BATCH, IN_FEATS)
  return _mlp(h, W1, b1.reshape(1, MLP_DIM), W2, b2.reshape(1, MLP_DIM),
              W3, b3.reshape(1, MLP_DIM), W4, b4.reshape(1, OUT_DIM))


# speedup vs baseline: 24.9970x; 1.2321x over previous
import functools

import jax
import jax.numpy as jnp
from jax import lax
from jax.experimental import pallas as pl
from jax.experimental.pallas import tpu as pltpu
from jax.experimental.pallas import tpu_sc as plsc

NUM_HEADS = 16
VOCAB = 65536
HF = 2
BATCH = 16384
MLP_DIM = 64
OUT_DIM = 3
IN_FEATS = NUM_HEADS * HF

NC = 2
NS = 16
NW = NC * NS
LANES = 16


def _sc_gather_body(table_hbm, idx_hbm, out_hbm, tbl_v, idx_v, out_v, sem):
  wid = lax.axis_index("s") * NC + lax.axis_index("c")
  head = wid // 2

  pltpu.sync_copy(table_hbm.at[wid], tbl_v)
  pltpu.sync_copy(idx_hbm.at[head], idx_v)

  @pl.loop(0, BATCH // LANES, unroll=8)
  def step(i):
    sl = pl.ds(i * LANES, LANES)
    out_v[sl] = plsc.load_gather(tbl_v, [idx_v[sl]])

  pltpu.sync_copy(out_v, out_hbm.at[wid])


_sc_gather = functools.partial(
    pl.kernel,
    out_type=jax.ShapeDtypeStruct((NW, BATCH), jnp.float32),
    mesh=plsc.VectorSubcoreMesh(core_axis_name="c", subcore_axis_name="s"),
    scratch_types=[
        pltpu.VMEM((VOCAB,), jnp.float32),
        pltpu.VMEM((BATCH,), jnp.int32),
        pltpu.VMEM((BATCH,), jnp.float32),
        pltpu.SemaphoreType.DMA,
    ],
    compiler_params=pltpu.CompilerParams(
        use_tc_tiling_on_sc=False, needs_layout_passes=False),
)(_sc_gather_body)


def _mlp_body(x_ref, w1t, b1, w2t, b2, w3t, b3, w4t, b4, o_ref):
  x = x_ref[...]
  x = jnp.maximum(
      jnp.dot(w1t[...], x, preferred_element_type=jnp.float32) + b1[...], 0.0)
  x = jnp.maximum(
      jnp.dot(w2t[...], x, preferred_element_type=jnp.float32) + b2[...], 0.0)
  x = jnp.maximum(
      jnp.dot(w3t[...], x, preferred_element_type=jnp.float32) + b3[...], 0.0)
  o_ref[...] = (
      jnp.dot(w4t[...], x, preferred_element_type=jnp.float32) + b4[...])


BBLK = 2048


def _mlp_t(emb, W1t, b1, W2t, b2, W3t, b3, W4t, b4):
  full = lambda i: (0, 0)
  return pl.pallas_call(
      _mlp_body,
      grid=(BATCH // BBLK,),
      in_specs=[
          pl.BlockSpec((IN_FEATS, BBLK), lambda i: (0, i)),
          pl.BlockSpec((MLP_DIM, IN_FEATS), full),
          pl.BlockSpec((MLP_DIM, 1), full),
          pl.BlockSpec((MLP_DIM, MLP_DIM), full),
          pl.BlockSpec((MLP_DIM, 1), full),
          pl.BlockSpec((MLP_DIM, MLP_DIM), full),
          pl.BlockSpec((MLP_DIM, 1), full),
          pl.BlockSpec((OUT_DIM, MLP_DIM), full),
          pl.BlockSpec((OUT_DIM, 1), full),
      ],
      out_specs=pl.BlockSpec((OUT_DIM, BBLK), lambda i: (0, i)),
      out_shape=jax.ShapeDtypeStruct((OUT_DIM, BATCH), jnp.float32),
  )(emb, W1t, b1, W2t, b2, W3t, b3, W4t, b4)


@jax.jit
def kernel(input, tables, W1, b1, W2, b2, W3, b3, W4, b4):
  tables_t = tables.transpose(0, 2, 1).reshape(NW, VOCAB)
  idx_t = input.T
  emb = _sc_gather(tables_t, idx_t)
  out_t = _mlp_t(emb, W1.T, b1.reshape(MLP_DIM, 1), W2.T,
                 b2.reshape(MLP_DIM, 1), W3.T, b3.reshape(MLP_DIM, 1),
                 W4.T, b4.reshape(OUT_DIM, 1))
  return out_t.T


# speedup vs baseline: 27.8083x; 1.1125x over previous
import functools

import jax
import jax.numpy as jnp
from jax import lax
from jax.experimental import pallas as pl
from jax.experimental.pallas import tpu as pltpu
from jax.experimental.pallas import tpu_sc as plsc

NUM_HEADS = 16
VOCAB = 65536
HF = 2
BATCH = 16384
MLP_DIM = 64
OUT_DIM = 3
IN_FEATS = NUM_HEADS * HF

NC = 2
NS = 16
NW = NC * NS
LANES = 16


def _sc_gather_body(table_hbm, idx_hbm, out_hbm, tbl_v, idx_v, out_v, sem):
  wid = lax.axis_index("s") * NC + lax.axis_index("c")
  head = wid // 2

  pltpu.sync_copy(table_hbm.at[wid], tbl_v)
  pltpu.sync_copy(idx_hbm.at[head], idx_v)

  @pl.loop(0, BATCH // LANES)
  def step(i):
    sl = pl.ds(i * LANES, LANES)
    out_v[sl] = plsc.load_gather(tbl_v, [idx_v[sl]])

  pltpu.sync_copy(out_v, out_hbm.at[wid])


_sc_gather = functools.partial(
    pl.kernel,
    out_type=jax.ShapeDtypeStruct((NW, BATCH), jnp.float32),
    mesh=plsc.VectorSubcoreMesh(core_axis_name="c", subcore_axis_name="s"),
    scratch_types=[
        pltpu.VMEM((VOCAB,), jnp.float32),
        pltpu.VMEM((BATCH,), jnp.int32),
        pltpu.VMEM((BATCH,), jnp.float32),
        pltpu.SemaphoreType.DMA,
    ],
    compiler_params=pltpu.CompilerParams(
        use_tc_tiling_on_sc=False, needs_layout_passes=False),
)(_sc_gather_body)


def _mlp_body(x_ref, w1t, b1, w2t, b2, w3t, b3, w4t, b4, o_ref):
  x = x_ref[...]
  x = jnp.maximum(
      jnp.dot(w1t[...], x, preferred_element_type=jnp.float32) + b1[...], 0.0)
  x = jnp.maximum(
      jnp.dot(w2t[...], x, preferred_element_type=jnp.float32) + b2[...], 0.0)
  x = jnp.maximum(
      jnp.dot(w3t[...], x, preferred_element_type=jnp.float32) + b3[...], 0.0)
  o_ref[...] = (
      jnp.dot(w4t[...], x, preferred_element_type=jnp.float32) + b4[...])


BBLK = 4096


def _mlp_t(emb, W1t, b1, W2t, b2, W3t, b3, W4t, b4):
  full = lambda i: (0, 0)
  return pl.pallas_call(
      _mlp_body,
      grid=(BATCH // BBLK,),
      in_specs=[
          pl.BlockSpec((IN_FEATS, BBLK), lambda i: (0, i)),
          pl.BlockSpec((MLP_DIM, IN_FEATS), full),
          pl.BlockSpec((MLP_DIM, 1), full),
          pl.BlockSpec((MLP_DIM, MLP_DIM), full),
          pl.BlockSpec((MLP_DIM, 1), full),
          pl.BlockSpec((MLP_DIM, MLP_DIM), full),
          pl.BlockSpec((MLP_DIM, 1), full),
          pl.BlockSpec((OUT_DIM, MLP_DIM), full),
          pl.BlockSpec((OUT_DIM, 1), full),
      ],
      out_specs=pl.BlockSpec((OUT_DIM, BBLK), lambda i: (0, i)),
      out_shape=jax.ShapeDtypeStruct((OUT_DIM, BATCH), jnp.float32),
  )(emb, W1t, b1, W2t, b2, W3t, b3, W4t, b4)


@jax.jit
def kernel(input, tables, W1, b1, W2, b2, W3, b3, W4, b4):
  tables_t = tables.transpose(0, 2, 1).reshape(NW, VOCAB)
  idx_t = input.T
  emb = _sc_gather(tables_t, idx_t)
  out_t = _mlp_t(emb, W1.T, b1.reshape(MLP_DIM, 1), W2.T,
                 b2.reshape(MLP_DIM, 1), W3.T, b3.reshape(MLP_DIM, 1),
                 W4.T, b4.reshape(OUT_DIM, 1))
  return out_t.T


# speedup vs baseline: 29.0568x; 1.0449x over previous
import functools

import jax
import jax.numpy as jnp
from jax import lax
from jax.experimental import pallas as pl
from jax.experimental.pallas import tpu as pltpu
from jax.experimental.pallas import tpu_sc as plsc

NUM_HEADS = 16
VOCAB = 65536
HF = 2
BATCH = 16384
MLP_DIM = 64
OUT_DIM = 3
IN_FEATS = NUM_HEADS * HF

NC = 2
NS = 16
NW = NC * NS
LANES = 16


def _sc_gather_body(table_hbm, idx_hbm, out_hbm, tbl_v, idx_v, out_v, sem):
  wid = lax.axis_index("s") * NC + lax.axis_index("c")
  head = wid // 2

  c1 = pltpu.async_copy(table_hbm.at[wid], tbl_v, sem)
  c2 = pltpu.async_copy(idx_hbm.at[head], idx_v, sem)
  c1.wait()
  c2.wait()

  HALF = BATCH // 2

  @pl.loop(0, HALF // LANES)
  def step(i):
    sl0 = pl.ds(i * LANES, LANES)
    sl1 = pl.ds(HALF + i * LANES, LANES)
    out_v[sl0] = plsc.load_gather(tbl_v, [idx_v[sl0]])
    out_v[sl1] = plsc.load_gather(tbl_v, [idx_v[sl1]])

  pltpu.sync_copy(out_v, out_hbm.at[wid])


_sc_gather = functools.partial(
    pl.kernel,
    out_type=jax.ShapeDtypeStruct((NW, BATCH), jnp.float32),
    mesh=plsc.VectorSubcoreMesh(core_axis_name="c", subcore_axis_name="s"),
    scratch_types=[
        pltpu.VMEM((VOCAB,), jnp.float32),
        pltpu.VMEM((BATCH,), jnp.int32),
        pltpu.VMEM((BATCH,), jnp.float32),
        pltpu.SemaphoreType.DMA,
    ],
    compiler_params=pltpu.CompilerParams(
        use_tc_tiling_on_sc=False, needs_layout_passes=False),
)(_sc_gather_body)


def _mlp_body(x_ref, w1t, b1, w2t, b2, w3t, b3, w4t, b4, o_ref):
  x = x_ref[...]
  x = jnp.maximum(
      jnp.dot(w1t[...], x, preferred_element_type=jnp.float32) + b1[...], 0.0)
  x = jnp.maximum(
      jnp.dot(w2t[...], x, preferred_element_type=jnp.float32) + b2[...], 0.0)
  x = jnp.maximum(
      jnp.dot(w3t[...], x, preferred_element_type=jnp.float32) + b3[...], 0.0)
  o_ref[...] = (
      jnp.dot(w4t[...], x, preferred_element_type=jnp.float32) + b4[...])


BBLK = 4096


def _mlp_t(emb, W1t, b1, W2t, b2, W3t, b3, W4t, b4):
  full = lambda i: (0, 0)
  return pl.pallas_call(
      _mlp_body,
      grid=(BATCH // BBLK,),
      in_specs=[
          pl.BlockSpec((IN_FEATS, BBLK), lambda i: (0, i)),
          pl.BlockSpec((MLP_DIM, IN_FEATS), full),
          pl.BlockSpec((MLP_DIM, 1), full),
          pl.BlockSpec((MLP_DIM, MLP_DIM), full),
          pl.BlockSpec((MLP_DIM, 1), full),
          pl.BlockSpec((MLP_DIM, MLP_DIM), full),
          pl.BlockSpec((MLP_DIM, 1), full),
          pl.BlockSpec((OUT_DIM, MLP_DIM), full),
          pl.BlockSpec((OUT_DIM, 1), full),
      ],
      out_specs=pl.BlockSpec((OUT_DIM, BBLK), lambda i: (0, i)),
      out_shape=jax.ShapeDtypeStruct((OUT_DIM, BATCH), jnp.float32),
  )(emb, W1t, b1, W2t, b2, W3t, b3, W4t, b4)


@jax.jit
def kernel(input, tables, W1, b1, W2, b2, W3, b3, W4, b4):
  tables_t = tables.transpose(0, 2, 1).reshape(NW, VOCAB)
  idx_t = input.T
  emb = _sc_gather(tables_t, idx_t)
  out_t = _mlp_t(emb, W1.T, b1.reshape(MLP_DIM, 1), W2.T,
                 b2.reshape(MLP_DIM, 1), W3.T, b3.reshape(MLP_DIM, 1),
                 W4.T, b4.reshape(OUT_DIM, 1))
  return out_t.T


# speedup vs baseline: 29.3148x; 1.0089x over previous
import functools

import jax
import jax.numpy as jnp
from jax import lax
from jax.experimental import pallas as pl
from jax.experimental.pallas import tpu as pltpu
from jax.experimental.pallas import tpu_sc as plsc

NUM_HEADS = 16
VOCAB = 65536
HF = 2
BATCH = 16384
MLP_DIM = 64
OUT_DIM = 3
IN_FEATS = NUM_HEADS * HF

NC = 2
NS = 16
NW = NC * NS
LANES = 16


def _sc_gather_body(table_hbm, idx_hbm, out_hbm, tbl_v, idx_v, out_v, sem):
  wid = lax.axis_index("s") * NC + lax.axis_index("c")
  head = wid // 2

  c1 = pltpu.async_copy(table_hbm.at[wid], tbl_v, sem)
  c2 = pltpu.async_copy(idx_hbm.at[head], idx_v, sem)
  c1.wait()
  c2.wait()

  HALF = BATCH // 2

  @pl.loop(0, HALF // LANES)
  def step(i):
    sl0 = pl.ds(i * LANES, LANES)
    sl1 = pl.ds(HALF + i * LANES, LANES)
    out_v[sl0] = plsc.load_gather(tbl_v, [idx_v[sl0]])
    out_v[sl1] = plsc.load_gather(tbl_v, [idx_v[sl1]])

  pltpu.sync_copy(out_v, out_hbm.at[wid])


_sc_gather = functools.partial(
    pl.kernel,
    out_type=jax.ShapeDtypeStruct((NW, BATCH), jnp.float32),
    mesh=plsc.VectorSubcoreMesh(core_axis_name="c", subcore_axis_name="s"),
    scratch_types=[
        pltpu.VMEM((VOCAB,), jnp.float32),
        pltpu.VMEM((BATCH,), jnp.int32),
        pltpu.VMEM((BATCH,), jnp.float32),
        pltpu.SemaphoreType.DMA,
    ],
    compiler_params=pltpu.CompilerParams(
        use_tc_tiling_on_sc=False, needs_layout_passes=False),
)(_sc_gather_body)


def _mlp_body(x_ref, w1t, b1, w2t, b2, w3t, b3, w4t, b4, o_ref):
  x = x_ref[...]
  x = jnp.maximum(
      jnp.dot(w1t[...], x, preferred_element_type=jnp.float32) + b1[...], 0.0)
  x = jnp.maximum(
      jnp.dot(w2t[...], x, preferred_element_type=jnp.float32) + b2[...], 0.0)
  x = jnp.maximum(
      jnp.dot(w3t[...], x, preferred_element_type=jnp.float32) + b3[...], 0.0)
  o_ref[...] = (
      jnp.dot(w4t[...], x, preferred_element_type=jnp.float32) + b4[...])


BBLK = 16384


def _mlp_t(emb, W1t, b1, W2t, b2, W3t, b3, W4t, b4):
  full = lambda i: (0, 0)
  return pl.pallas_call(
      _mlp_body,
      grid=(BATCH // BBLK,),
      in_specs=[
          pl.BlockSpec((IN_FEATS, BBLK), lambda i: (0, i)),
          pl.BlockSpec((MLP_DIM, IN_FEATS), full),
          pl.BlockSpec((MLP_DIM, 1), full),
          pl.BlockSpec((MLP_DIM, MLP_DIM), full),
          pl.BlockSpec((MLP_DIM, 1), full),
          pl.BlockSpec((MLP_DIM, MLP_DIM), full),
          pl.BlockSpec((MLP_DIM, 1), full),
          pl.BlockSpec((OUT_DIM, MLP_DIM), full),
          pl.BlockSpec((OUT_DIM, 1), full),
      ],
      out_specs=pl.BlockSpec((OUT_DIM, BBLK), lambda i: (0, i)),
      out_shape=jax.ShapeDtypeStruct((OUT_DIM, BATCH), jnp.float32),
  )(emb, W1t, b1, W2t, b2, W3t, b3, W4t, b4)


@jax.jit
def kernel(input, tables, W1, b1, W2, b2, W3, b3, W4, b4):
  tables_t = tables.transpose(0, 2, 1).reshape(NW, VOCAB)
  idx_t = input.T
  emb = _sc_gather(tables_t, idx_t)
  out_t = _mlp_t(emb, W1.T, b1.reshape(MLP_DIM, 1), W2.T,
                 b2.reshape(MLP_DIM, 1), W3.T, b3.reshape(MLP_DIM, 1),
                 W4.T, b4.reshape(OUT_DIM, 1))
  return out_t.T
